# baseline (speedup 1.0000x reference)
import functools

import jax
import jax.numpy as jnp
from jax import lax
from jax.experimental import pallas as pl
from jax.experimental.pallas import tpu as pltpu
from jax.experimental.pallas import tpu_sc as plsc

B = 16384
D = 128
N_GENRE = 18
DENSE_IN = 768

NC = 2
NS = 16
NW = NC * NS
BPW = B // NW
CHUNK = 128
NCHUNK = BPW // CHUNK


def _sc_gather(id_idx2d, id_table):
    mesh = plsc.VectorSubcoreMesh(
        core_axis_name="c", subcore_axis_name="s", num_cores=NC, num_subcores=NS
    )

    @functools.partial(
        pl.kernel,
        out_type=jax.ShapeDtypeStruct((B, 4 * D), jnp.float32),
        mesh=mesh,
        scratch_types=[
            pltpu.VMEM((NCHUNK, CHUNK), jnp.int32),
            pltpu.VMEM((NCHUNK, CHUNK, D), jnp.float32),
            pltpu.SemaphoreType.DMA,
            pltpu.SemaphoreType.DMA,
        ],
    )
    def body(id_hbm, idtab_hbm, out, idx_id, ring, gsem, wsem):
        wid = lax.axis_index("s") * NC + lax.axis_index("c")
        base = wid * BPW
        row_base = wid * NCHUNK

        pltpu.sync_copy(id_hbm.at[pl.ds(row_base, NCHUNK)], idx_id)

        def out_slice(j):
            return out.at[pl.ds(base + j * CHUNK, CHUNK), pl.ds(0, D)]

        for j in range(NCHUNK):
            pltpu.async_copy(idtab_hbm.at[idx_id.at[j]], ring.at[j], gsem)
        for j in range(NCHUNK):
            pltpu.make_async_copy(idtab_hbm.at[idx_id.at[j]], ring.at[j], gsem).wait()
            pltpu.async_copy(ring.at[j], out_slice(j), wsem)
        for j in range(NCHUNK):
            pltpu.make_async_copy(ring.at[j], out_slice(j), wsem).wait()

    return body(id_idx2d, id_table)


def _tc_body(bb, alias_ref, date_ref, g_ref, t_ref, dtab_ref, gm_ref, w_ref,
             b_ref, o_ref):
    i = pl.program_id(0)
    j = pl.program_id(1)

    @pl.when(j == 0)
    def _date():
        date_blk = date_ref[pl.ds(i * bb, bb), :]
        lanes = lax.broadcasted_iota(jnp.int32, (bb, D), 1)
        one_hot = (date_blk == lanes).astype(jnp.float32)
        o_ref[...] = jnp.dot(
            one_hot, dtab_ref[...], preferred_element_type=jnp.float32,
            precision=lax.Precision.HIGHEST,
        )

    @pl.when(j == 1)
    def _genre():
        g_blk = g_ref[pl.ds(i * bb, bb), :]
        o_ref[...] = jnp.dot(
            g_blk, gm_ref[...], preferred_element_type=jnp.float32
        )

    @pl.when(j == 2)
    def _dense():
        o_ref[...] = (
            jnp.dot(t_ref[...], w_ref[...], preferred_element_type=jnp.float32)
            + b_ref[...]
        )


def kernel(id, date, genres, title_embedding, id_table, date_table,
           genre_embedding_matrix, W_dense, b_dense):
    id2d = id.astype(jnp.int32).reshape(NW * NCHUNK, CHUNK)

    sc_out = _sc_gather(id2d, id_table)

    dtab_pad = jnp.zeros((D, D), jnp.float32).at[:100, :].set(date_table)

    bb = 2048
    out = pl.pallas_call(
        functools.partial(_tc_body, bb),
        grid=(B // bb, 3),
        in_specs=[
            pl.BlockSpec(memory_space=pl.ANY),
            pl.BlockSpec((B, 1), lambda i, j: (0, 0)),
            pl.BlockSpec((B, N_GENRE), lambda i, j: (0, 0)),
            pl.BlockSpec((bb, DENSE_IN), lambda i, j: (i, 0)),
            pl.BlockSpec((D, D), lambda i, j: (0, 0)),
            pl.BlockSpec((N_GENRE, D), lambda i, j: (0, 0)),
            pl.BlockSpec((DENSE_IN, D), lambda i, j: (0, 0)),
            pl.BlockSpec((1, D), lambda i, j: (0, 0)),
        ],
        out_specs=pl.BlockSpec((bb, D), lambda i, j: (i, j + 1)),
        out_shape=jax.ShapeDtypeStruct((B, 4 * D), jnp.float32),
        input_output_aliases={0: 0},
    )(sc_out, date.astype(jnp.int32).reshape(B, 1), genres, title_embedding,
      dtab_pad, genre_embedding_matrix, W_dense, b_dense.reshape(1, D))
    return out


# speedup vs baseline: 1.0602x; 1.0602x over previous
import functools

import jax
import jax.numpy as jnp
from jax import lax
from jax.experimental import pallas as pl
from jax.experimental.pallas import tpu as pltpu
from jax.experimental.pallas import tpu_sc as plsc

B = 16384
D = 128
N_GENRE = 18
DENSE_IN = 768

NC = 2
NS = 16
NW = NC * NS
BPW = B // NW
CHUNK = 128
NCHUNK = BPW // CHUNK


def _sc_gather(id_idx2d, id_table):
    mesh = plsc.VectorSubcoreMesh(
        core_axis_name="c", subcore_axis_name="s", num_cores=NC, num_subcores=NS
    )

    @functools.partial(
        pl.kernel,
        out_type=jax.ShapeDtypeStruct((B, 4 * D), jnp.float32),
        mesh=mesh,
        scratch_types=[
            pltpu.VMEM((NCHUNK, CHUNK), jnp.int32),
            pltpu.VMEM((NCHUNK, CHUNK, D), jnp.float32),
            pltpu.SemaphoreType.DMA,
            pltpu.SemaphoreType.DMA,
        ],
    )
    def body(id_hbm, idtab_hbm, out, idx_id, ring, gsem, wsem):
        wid = lax.axis_index("s") * NC + lax.axis_index("c")
        base = wid * BPW
        row_base = wid * NCHUNK

        pltpu.sync_copy(id_hbm.at[pl.ds(row_base, NCHUNK)], idx_id)

        def out_slice(j):
            return out.at[pl.ds(base + j * CHUNK, CHUNK), pl.ds(0, D)]

        for j in range(NCHUNK):
            pltpu.async_copy(idtab_hbm.at[idx_id.at[j]], ring.at[j], gsem)
        for j in range(NCHUNK):
            pltpu.make_async_copy(idtab_hbm.at[idx_id.at[j]], ring.at[j], gsem).wait()
            pltpu.async_copy(ring.at[j], out_slice(j), wsem)
        for j in range(NCHUNK):
            pltpu.make_async_copy(ring.at[j], out_slice(j), wsem).wait()

    return body(id_idx2d, id_table)


def _tc_body(bb, alias_ref, date_ref, g_ref, t_ref, dtab_ref, gm_ref, w_ref,
             b_ref, o_ref):
    i = pl.program_id(0)
    j = pl.program_id(1)

    @pl.when(j == 0)
    def _date():
        date_blk = date_ref[pl.ds(i * bb, bb), :]
        lanes = lax.broadcasted_iota(jnp.int32, (bb, D), 1)
        one_hot = (date_blk == lanes).astype(jnp.float32)
        o_ref[...] = jnp.dot(
            one_hot, dtab_ref[...], preferred_element_type=jnp.float32
        )

    @pl.when(j == 1)
    def _genre():
        g_blk = g_ref[pl.ds(i * bb, bb), :]
        o_ref[...] = jnp.dot(
            g_blk, gm_ref[...], preferred_element_type=jnp.float32
        )

    @pl.when(j == 2)
    def _dense():
        o_ref[...] = (
            jnp.dot(t_ref[...], w_ref[...], preferred_element_type=jnp.float32)
            + b_ref[...]
        )


def kernel(id, date, genres, title_embedding, id_table, date_table,
           genre_embedding_matrix, W_dense, b_dense):
    id2d = id.astype(jnp.int32).reshape(NW * NCHUNK, CHUNK)

    sc_out = _sc_gather(id2d, id_table)

    dtab_pad = jnp.zeros((D, D), jnp.float32).at[:100, :].set(date_table)

    bb = 2048
    out = pl.pallas_call(
        functools.partial(_tc_body, bb),
        grid=(B // bb, 3),
        in_specs=[
            pl.BlockSpec(memory_space=pl.ANY),
            pl.BlockSpec((B, 1), lambda i, j: (0, 0)),
            pl.BlockSpec((B, N_GENRE), lambda i, j: (0, 0)),
            pl.BlockSpec((bb, DENSE_IN), lambda i, j: (i, 0)),
            pl.BlockSpec((D, D), lambda i, j: (0, 0)),
            pl.BlockSpec((N_GENRE, D), lambda i, j: (0, 0)),
            pl.BlockSpec((DENSE_IN, D), lambda i, j: (0, 0)),
            pl.BlockSpec((1, D), lambda i, j: (0, 0)),
        ],
        out_specs=pl.BlockSpec((bb, D), lambda i, j: (i, j + 1)),
        out_shape=jax.ShapeDtypeStruct((B, 4 * D), jnp.float32),
        input_output_aliases={0: 0},
    )(sc_out, date.astype(jnp.int32).reshape(B, 1), genres, title_embedding,
      dtab_pad, genre_embedding_matrix, W_dense, b_dense.reshape(1, D))
    return out


# speedup vs baseline: 1.1868x; 1.1194x over previous
import functools

import jax
import jax.numpy as jnp
from jax import lax
from jax.experimental import pallas as pl
from jax.experimental.pallas import tpu as pltpu
from jax.experimental.pallas import tpu_sc as plsc

B = 16384
D = 128
N_GENRE = 18
DENSE_IN = 768

NC = 2
NS = 16
NW = NC * NS
BPW = B // NW
CHUNK = 128
NCHUNK = BPW // CHUNK


def _sc_gather(id_idx2d, id_table):
    mesh = plsc.VectorSubcoreMesh(
        core_axis_name="c", subcore_axis_name="s", num_cores=NC, num_subcores=NS
    )

    @functools.partial(
        pl.kernel,
        out_type=jax.ShapeDtypeStruct((B, D), jnp.float32),
        mesh=mesh,
        scratch_types=[
            pltpu.VMEM((NCHUNK, CHUNK), jnp.int32),
            pltpu.VMEM((NCHUNK, CHUNK, D), jnp.float32),
            pltpu.SemaphoreType.DMA,
            pltpu.SemaphoreType.DMA,
        ],
    )
    def body(id_hbm, idtab_hbm, out, idx_id, ring, gsem, wsem):
        wid = lax.axis_index("s") * NC + lax.axis_index("c")
        base = wid * BPW
        row_base = wid * NCHUNK

        pltpu.sync_copy(id_hbm.at[pl.ds(row_base, NCHUNK)], idx_id)

        for j in range(NCHUNK):
            pltpu.async_copy(idtab_hbm.at[idx_id.at[j]], ring.at[j], gsem)
        for j in range(NCHUNK):
            pltpu.make_async_copy(idtab_hbm.at[idx_id.at[j]], ring.at[j], gsem).wait()
            pltpu.async_copy(
                ring.at[j], out.at[pl.ds(base + j * CHUNK, CHUNK)], wsem
            )
        for j in range(NCHUNK):
            pltpu.make_async_copy(
                ring.at[j], out.at[pl.ds(base + j * CHUNK, CHUNK)], wsem
            ).wait()

    return body(id_idx2d, id_table)


def _tc_body(bb, id_ref, date_ref, g_ref, t_ref, dtab_ref, gm_ref, w_ref,
             b_ref, o_ref):
    i = pl.program_id(0)
    o_ref[:, 0:D] = id_ref[...]
    date_blk = date_ref[pl.ds(i * bb, bb), :]
    lanes = lax.broadcasted_iota(jnp.int32, (bb, D), 1)
    one_hot = (date_blk == lanes).astype(jnp.float32)
    o_ref[:, D:2 * D] = jnp.dot(
        one_hot, dtab_ref[...], preferred_element_type=jnp.float32
    )
    g_blk = g_ref[pl.ds(i * bb, bb), :]
    o_ref[:, 2 * D:3 * D] = jnp.dot(
        g_blk, gm_ref[...], preferred_element_type=jnp.float32
    )
    o_ref[:, 3 * D:4 * D] = (
        jnp.dot(t_ref[...], w_ref[...], preferred_element_type=jnp.float32)
        + b_ref[...]
    )


def kernel(id, date, genres, title_embedding, id_table, date_table,
           genre_embedding_matrix, W_dense, b_dense):
    id2d = id.astype(jnp.int32).reshape(NW * NCHUNK, CHUNK)

    id_emb = _sc_gather(id2d, id_table)

    dtab_pad = jnp.zeros((D, D), jnp.float32).at[:100, :].set(date_table)

    bb = 2048
    out = pl.pallas_call(
        functools.partial(_tc_body, bb),
        grid=(B // bb,),
        in_specs=[
            pl.BlockSpec((bb, D), lambda i: (i, 0)),
            pl.BlockSpec((B, 1), lambda i: (0, 0)),
            pl.BlockSpec((B, N_GENRE), lambda i: (0, 0)),
            pl.BlockSpec((bb, DENSE_IN), lambda i: (i, 0)),
            pl.BlockSpec((D, D), lambda i: (0, 0)),
            pl.BlockSpec((N_GENRE, D), lambda i: (0, 0)),
            pl.BlockSpec((DENSE_IN, D), lambda i: (0, 0)),
            pl.BlockSpec((1, D), lambda i: (0, 0)),
        ],
        out_specs=pl.BlockSpec((bb, 4 * D), lambda i: (i, 0)),
        out_shape=jax.ShapeDtypeStruct((B, 4 * D), jnp.float32),
    )(id_emb, date.astype(jnp.int32).reshape(B, 1), genres, title_embedding,
      dtab_pad, genre_embedding_matrix, W_dense, b_dense.reshape(1, D))
    return out
